# Initial kernel scaffold; baseline (speedup 1.0000x reference)
import jax
import jax.numpy as jnp
from jax.experimental import pallas as pl


def kernel(x, gate_w, w1, w3, w2):
    raise NotImplementedError("write your pallas kernel here")


# speedup vs baseline: 1.0463x; 1.0463x over previous
import functools

import jax
import jax.numpy as jnp
from jax.experimental import pallas as pl
from jax.experimental.pallas import tpu as pltpu

E = 8
TOPK = 2
D = 2048
F = 1408
T = 8192

BT = 512
BF = 128
NF = F // BF
NP = T * TOPK + E * BT
NB = NP // BT


def _mlp_body(be_ref, xs_ref, rw_ref, w1_ref, w3_ref, w2_ref, ys_ref, h_ref):
    f = pl.program_id(1)
    x = xs_ref[...]
    g = jnp.dot(x, w1_ref[0], preferred_element_type=jnp.float32)
    u = jnp.dot(x, w3_ref[0], preferred_element_type=jnp.float32)
    off = pl.multiple_of(f * BF, BF)
    h_ref[:, pl.ds(off, BF)] = (g * jax.nn.sigmoid(g)) * u

    @pl.when(f == NF - 1)
    def _down():
        ys_ref[...] = jnp.dot(
            h_ref[...], w2_ref[0], preferred_element_type=jnp.float32
        ) * rw_ref[...]


_grouped_mlp = pl.pallas_call(
    _mlp_body,
    grid_spec=pltpu.PrefetchScalarGridSpec(
        num_scalar_prefetch=1,
        grid=(NB, NF),
        in_specs=[
            pl.BlockSpec((BT, D), lambda b, f, be: (b, 0)),
            pl.BlockSpec((BT, 1), lambda b, f, be: (b, 0)),
            pl.BlockSpec((1, D, BF), lambda b, f, be: (be[b], 0, f)),
            pl.BlockSpec((1, D, BF), lambda b, f, be: (be[b], 0, f)),
            pl.BlockSpec((1, F, D), lambda b, f, be: (be[b], 0, 0)),
        ],
        out_specs=pl.BlockSpec((BT, D), lambda b, f, be: (b, 0)),
        scratch_shapes=[pltpu.VMEM((BT, F), jnp.float32)],
    ),
    out_shape=jax.ShapeDtypeStruct((NP, D), jnp.float32),
    compiler_params=pltpu.CompilerParams(
        dimension_semantics=("arbitrary", "arbitrary")),
)


def kernel(x, gate_w, w1, w3, w2):
    logits = x @ gate_w
    probs = jax.nn.softmax(logits, axis=-1)
    tw, ti = jax.lax.top_k(probs, TOPK)
    tw = tw / jnp.sum(tw, axis=-1, keepdims=True)
    e0, e1 = ti[:, 0], ti[:, 1]

    oh = (jax.nn.one_hot(e0, E, dtype=jnp.int32)
          + jax.nn.one_hot(e1, E, dtype=jnp.int32))
    cinc = jnp.cumsum(oh, axis=0)
    cexc = cinc - oh
    total = cinc[-1]
    padded = ((total + BT - 1) // BT) * BT
    ends = jnp.cumsum(padded)
    base = ends - padded
    tarange = jnp.arange(T)
    pos0 = base[e0] + cexc[tarange, e0]
    pos1 = base[e1] + cexc[tarange, e1]

    token_src = (jnp.zeros((NP,), jnp.int32)
                 .at[pos0].set(tarange)
                 .at[pos1].set(tarange))
    row_w = (jnp.zeros((NP,), jnp.float32)
             .at[pos0].set(tw[:, 0])
             .at[pos1].set(tw[:, 1]))
    block_expert = jnp.minimum(
        jnp.searchsorted(ends, jnp.arange(NB, dtype=jnp.int32) * BT,
                         side="right"),
        E - 1).astype(jnp.int32)

    xs = x[token_src]
    ys = _grouped_mlp(block_expert, xs, row_w[:, None], w1, w3, w2)
    out = ys[pos0] + ys[pos1]
    return out


# speedup vs baseline: 1.3680x; 1.3075x over previous
import functools

import jax
import jax.numpy as jnp
from jax import lax
from jax.experimental import pallas as pl
from jax.experimental.pallas import tpu as pltpu
from jax.experimental.pallas import tpu_sc as plsc

E = 8
TOPK = 2
D = 2048
F = 1408
T = 8192

BT = 512
BF = 128
NF = F // BF
NP = T * TOPK + E * BT
NB = NP // BT


def _mlp_body(be_ref, xs_ref, w1_ref, w3_ref, w2_ref, ys_ref, h_ref):
    f = pl.program_id(1)
    x = xs_ref[...]
    g = jnp.dot(x, w1_ref[0], preferred_element_type=jnp.float32)
    u = jnp.dot(x, w3_ref[0], preferred_element_type=jnp.float32)
    off = pl.multiple_of(f * BF, BF)
    h_ref[:, pl.ds(off, BF)] = (g * jax.nn.sigmoid(g)) * u

    @pl.when(f == NF - 1)
    def _down():
        ys_ref[...] = jnp.dot(
            h_ref[...], w2_ref[0], preferred_element_type=jnp.float32)


_grouped_mlp = pl.pallas_call(
    _mlp_body,
    grid_spec=pltpu.PrefetchScalarGridSpec(
        num_scalar_prefetch=1,
        grid=(NB, NF),
        in_specs=[
            pl.BlockSpec((BT, D), lambda b, f, be: (b, 0)),
            pl.BlockSpec((1, D, BF), lambda b, f, be: (be[b], 0, f)),
            pl.BlockSpec((1, D, BF), lambda b, f, be: (be[b], 0, f)),
            pl.BlockSpec((1, F, D), lambda b, f, be: (be[b], 0, 0)),
        ],
        out_specs=pl.BlockSpec((BT, D), lambda b, f, be: (b, 0)),
        scratch_shapes=[pltpu.VMEM((BT, F), jnp.float32)],
    ),
    out_shape=jax.ShapeDtypeStruct((NP, D), jnp.float32),
    compiler_params=pltpu.CompilerParams(
        dimension_semantics=("arbitrary", "arbitrary")),
)

_NW = 32
_TPW = T // _NW

_ACH = 16
_NCH = _TPW // _ACH

_SC_MESH = plsc.VectorSubcoreMesh(core_axis_name="c", subcore_axis_name="s")


@functools.partial(
    pl.kernel,
    mesh=_SC_MESH,
    out_type=jax.ShapeDtypeStruct((NP, D), jnp.float32),
    scratch_types=[
        pltpu.VMEM((_NCH, _ACH), jnp.int32),
        pltpu.VMEM((_NCH, _ACH), jnp.int32),
        pltpu.VMEM((_ACH, D), jnp.float32),
        pltpu.SemaphoreType.DMA,
    ],
)
def _sc_scatter_x(x_hbm, pos0_hbm, pos1_hbm, xs_hbm, p0_v, p1_v, xbuf, sem):
    wid = lax.axis_index("s") * 2 + lax.axis_index("c")
    tok0 = wid * _TPW
    pltpu.sync_copy(pos0_hbm.at[wid], p0_v)
    pltpu.sync_copy(pos1_hbm.at[wid], p1_v)

    def chunk(c, carry):
        pltpu.sync_copy(x_hbm.at[pl.ds(tok0 + c * _ACH, _ACH)], xbuf)
        cp0 = pltpu.async_copy(xbuf, xs_hbm.at[p0_v.at[c]], sem)
        cp1 = pltpu.async_copy(xbuf, xs_hbm.at[p1_v.at[c]], sem)
        cp0.wait()
        cp1.wait()
        return carry

    lax.fori_loop(0, _NCH, chunk, 0)


_BCH = 16
_NBC = _TPW // _BCH


@functools.partial(
    pl.kernel,
    mesh=_SC_MESH,
    out_type=jax.ShapeDtypeStruct((T, D), jnp.float32),
    scratch_types=[
        pltpu.VMEM((_NBC, _BCH), jnp.int32),
        pltpu.VMEM((_NBC, _BCH), jnp.int32),
        pltpu.VMEM((_BCH, 16), jnp.float32),
        pltpu.VMEM((_BCH, 16), jnp.float32),
        pltpu.VMEM((_BCH, D), jnp.float32),
        pltpu.VMEM((_BCH, D), jnp.float32),
        pltpu.SemaphoreType.DMA,
    ],
)
def _sc_combine(ys_hbm, pos0_hbm, pos1_hbm, tw0_hbm, tw1_hbm, out_hbm,
                p0_v, p1_v, w0_v, w1_v, abuf, bbuf, sem):
    wid = lax.axis_index("s") * 2 + lax.axis_index("c")
    tok0 = wid * _TPW
    pltpu.sync_copy(pos0_hbm.at[wid], p0_v)
    pltpu.sync_copy(pos1_hbm.at[wid], p1_v)

    def chunk(c, carry):
        cpa = pltpu.async_copy(ys_hbm.at[p0_v.at[c]], abuf, sem)
        cpb = pltpu.async_copy(ys_hbm.at[p1_v.at[c]], bbuf, sem)
        pltpu.sync_copy(tw0_hbm.at[pl.ds(tok0 + c * _BCH, _BCH)], w0_v)
        pltpu.sync_copy(tw1_hbm.at[pl.ds(tok0 + c * _BCH, _BCH)], w1_v)
        cpa.wait()
        cpb.wait()

        def row(r, carry2):
            w0 = w0_v[r]
            w1 = w1_v[r]

            def col(j, carry3):
                sl = pl.ds(j * 16, 16)
                abuf[r, sl] = w0 * abuf[r, sl] + w1 * bbuf[r, sl]
                return carry3

            return lax.fori_loop(0, D // 16, col, carry2)

        lax.fori_loop(0, _BCH, row, carry)
        pltpu.sync_copy(abuf, out_hbm.at[pl.ds(tok0 + c * _BCH, _BCH)])
        return carry

    lax.fori_loop(0, _NBC, chunk, 0)


def kernel(x, gate_w, w1, w3, w2):
    logits = x @ gate_w
    probs = jax.nn.softmax(logits, axis=-1)
    tw, ti = jax.lax.top_k(probs, TOPK)
    tw = tw / jnp.sum(tw, axis=-1, keepdims=True)
    e0, e1 = ti[:, 0], ti[:, 1]

    oh = (jax.nn.one_hot(e0, E, dtype=jnp.int32)
          + jax.nn.one_hot(e1, E, dtype=jnp.int32))
    cinc = jnp.cumsum(oh, axis=0)
    cexc = cinc - oh
    total = cinc[-1]
    padded = ((total + BT - 1) // BT) * BT
    ends = jnp.cumsum(padded)
    base = ends - padded
    tarange = jnp.arange(T)
    pos0 = base[e0] + cexc[tarange, e0]
    pos1 = base[e1] + cexc[tarange, e1]

    block_expert = jnp.minimum(
        jnp.searchsorted(ends, jnp.arange(NB, dtype=jnp.int32) * BT,
                         side="right"),
        E - 1).astype(jnp.int32)

    pos0 = pos0.astype(jnp.int32)
    pos1 = pos1.astype(jnp.int32)
    xs = _sc_scatter_x(x,
                       pos0.reshape(_NW, _NCH, _ACH),
                       pos1.reshape(_NW, _NCH, _ACH))
    ys = _grouped_mlp(block_expert, xs, w1, w3, w2)
    tw0r = jnp.broadcast_to(tw[:, 0:1], (T, 16))
    tw1r = jnp.broadcast_to(tw[:, 1:2], (T, 16))
    out = _sc_combine(ys,
                      pos0.reshape(_NW, _NBC, _BCH),
                      pos1.reshape(_NW, _NBC, _BCH),
                      tw0r, tw1r)
    return out


# speedup vs baseline: 1.4537x; 1.0626x over previous
import functools

import jax
import jax.numpy as jnp
from jax import lax
from jax.experimental import pallas as pl
from jax.experimental.pallas import tpu as pltpu
from jax.experimental.pallas import tpu_sc as plsc

E = 8
TOPK = 2
D = 2048
F = 1408
T = 8192

BT = 512
BD = 256
NDC = D // BD
NP = T * TOPK + E * BT
NB = NP // BT


def _mlp_body(be_ref, xs_ref, w1_ref, w3_ref, w2_ref, ys_ref, g_ref, u_ref):
    dstep = pl.program_id(1)
    x = xs_ref[...]
    pg = jnp.dot(x, w1_ref[0], preferred_element_type=jnp.float32)
    pu = jnp.dot(x, w3_ref[0], preferred_element_type=jnp.float32)

    @pl.when(dstep == 0)
    def _init():
        g_ref[...] = pg
        u_ref[...] = pu

    @pl.when(dstep != 0)
    def _acc():
        g_ref[...] += pg
        u_ref[...] += pu

    @pl.when(dstep == NDC - 1)
    def _down():
        g = g_ref[...]
        h = (g * jax.nn.sigmoid(g)) * u_ref[...]
        ys_ref[...] = jnp.dot(h, w2_ref[0], preferred_element_type=jnp.float32)


_grouped_mlp = pl.pallas_call(
    _mlp_body,
    grid_spec=pltpu.PrefetchScalarGridSpec(
        num_scalar_prefetch=1,
        grid=(NB, NDC),
        in_specs=[
            pl.BlockSpec((BT, BD), lambda b, d, be: (b, d)),
            pl.BlockSpec((1, BD, F), lambda b, d, be: (be[b], d, 0)),
            pl.BlockSpec((1, BD, F), lambda b, d, be: (be[b], d, 0)),
            pl.BlockSpec((1, F, D), lambda b, d, be: (be[b], 0, 0)),
        ],
        out_specs=pl.BlockSpec((BT, D), lambda b, d, be: (b, 0)),
        scratch_shapes=[pltpu.VMEM((BT, F), jnp.float32),
                        pltpu.VMEM((BT, F), jnp.float32)],
    ),
    out_shape=jax.ShapeDtypeStruct((NP, D), jnp.float32),
    compiler_params=pltpu.CompilerParams(
        dimension_semantics=("arbitrary", "arbitrary")),
)

_NW = 32
_TPW = T // _NW

_ACH = 16
_NCH = _TPW // _ACH

_SC_MESH = plsc.VectorSubcoreMesh(core_axis_name="c", subcore_axis_name="s")


@functools.partial(
    pl.kernel,
    mesh=_SC_MESH,
    out_type=jax.ShapeDtypeStruct((NP, D), jnp.float32),
    scratch_types=[
        pltpu.VMEM((_NCH, _ACH), jnp.int32),
        pltpu.VMEM((_NCH, _ACH), jnp.int32),
        pltpu.VMEM((_ACH, D), jnp.float32),
        pltpu.SemaphoreType.DMA,
    ],
)
def _sc_scatter_x(x_hbm, pos0_hbm, pos1_hbm, xs_hbm, p0_v, p1_v, xbuf, sem):
    wid = lax.axis_index("s") * 2 + lax.axis_index("c")
    tok0 = wid * _TPW
    pltpu.sync_copy(pos0_hbm.at[wid], p0_v)
    pltpu.sync_copy(pos1_hbm.at[wid], p1_v)

    def chunk(c, carry):
        pltpu.sync_copy(x_hbm.at[pl.ds(tok0 + c * _ACH, _ACH)], xbuf)
        cp0 = pltpu.async_copy(xbuf, xs_hbm.at[p0_v.at[c]], sem)
        cp1 = pltpu.async_copy(xbuf, xs_hbm.at[p1_v.at[c]], sem)
        cp0.wait()
        cp1.wait()
        return carry

    lax.fori_loop(0, _NCH, chunk, 0)


_BCH = 16
_NBC = _TPW // _BCH


@functools.partial(
    pl.kernel,
    mesh=_SC_MESH,
    out_type=jax.ShapeDtypeStruct((T, D), jnp.float32),
    scratch_types=[
        pltpu.VMEM((_NBC, _BCH), jnp.int32),
        pltpu.VMEM((_NBC, _BCH), jnp.int32),
        pltpu.VMEM((_BCH, 16), jnp.float32),
        pltpu.VMEM((_BCH, 16), jnp.float32),
        pltpu.VMEM((_BCH, D), jnp.float32),
        pltpu.VMEM((_BCH, D), jnp.float32),
        pltpu.SemaphoreType.DMA,
    ],
)
def _sc_combine(ys_hbm, pos0_hbm, pos1_hbm, tw0_hbm, tw1_hbm, out_hbm,
                p0_v, p1_v, w0_v, w1_v, abuf, bbuf, sem):
    wid = lax.axis_index("s") * 2 + lax.axis_index("c")
    tok0 = wid * _TPW
    pltpu.sync_copy(pos0_hbm.at[wid], p0_v)
    pltpu.sync_copy(pos1_hbm.at[wid], p1_v)

    def chunk(c, carry):
        cpa = pltpu.async_copy(ys_hbm.at[p0_v.at[c]], abuf, sem)
        cpb = pltpu.async_copy(ys_hbm.at[p1_v.at[c]], bbuf, sem)
        pltpu.sync_copy(tw0_hbm.at[pl.ds(tok0 + c * _BCH, _BCH)], w0_v)
        pltpu.sync_copy(tw1_hbm.at[pl.ds(tok0 + c * _BCH, _BCH)], w1_v)
        cpa.wait()
        cpb.wait()

        def row(r, carry2):
            w0 = w0_v[r]
            w1 = w1_v[r]

            def col(j, carry3):
                sl = pl.ds(j * 16, 16)
                abuf[r, sl] = w0 * abuf[r, sl] + w1 * bbuf[r, sl]
                return carry3

            return lax.fori_loop(0, D // 16, col, carry2)

        lax.fori_loop(0, _BCH, row, carry)
        pltpu.sync_copy(abuf, out_hbm.at[pl.ds(tok0 + c * _BCH, _BCH)])
        return carry

    lax.fori_loop(0, _NBC, chunk, 0)


def kernel(x, gate_w, w1, w3, w2):
    logits = x @ gate_w
    probs = jax.nn.softmax(logits, axis=-1)
    tw, ti = jax.lax.top_k(probs, TOPK)
    tw = tw / jnp.sum(tw, axis=-1, keepdims=True)
    e0, e1 = ti[:, 0], ti[:, 1]

    oh = (jax.nn.one_hot(e0, E, dtype=jnp.int32)
          + jax.nn.one_hot(e1, E, dtype=jnp.int32))
    cinc = jnp.cumsum(oh, axis=0)
    cexc = cinc - oh
    total = cinc[-1]
    padded = ((total + BT - 1) // BT) * BT
    ends = jnp.cumsum(padded)
    base = ends - padded
    tarange = jnp.arange(T)
    pos0 = base[e0] + cexc[tarange, e0]
    pos1 = base[e1] + cexc[tarange, e1]

    block_expert = jnp.minimum(
        jnp.searchsorted(ends, jnp.arange(NB, dtype=jnp.int32) * BT,
                         side="right"),
        E - 1).astype(jnp.int32)

    pos0 = pos0.astype(jnp.int32)
    pos1 = pos1.astype(jnp.int32)
    xs = _sc_scatter_x(x,
                       pos0.reshape(_NW, _NCH, _ACH),
                       pos1.reshape(_NW, _NCH, _ACH))
    ys = _grouped_mlp(block_expert, xs, w1, w3, w2)
    tw0r = jnp.broadcast_to(tw[:, 0:1], (T, 16))
    tw1r = jnp.broadcast_to(tw[:, 1:2], (T, 16))
    out = _sc_combine(ys,
                      pos0.reshape(_NW, _NBC, _BCH),
                      pos1.reshape(_NW, _NBC, _BCH),
                      tw0r, tw1r)
    return out
